# Initial kernel scaffold; baseline (speedup 1.0000x reference)
import jax
import jax.numpy as jnp
from jax.experimental import pallas as pl


def kernel(word_id_a, img_emb_a, edge_side_a, edge_upd_a, query_gid_a, word_id_p, img_emb_p, edge_side_p, edge_upd_p, query_gid_p, click_reverse, wordemb, trans_W, trans_b, W1_side, b1_side, W1_upd, b1_upd, W2_upd, b2_upd, D_W0, D_b0, D_W1, D_b1):
    raise NotImplementedError("write your pallas kernel here")


# speedup vs baseline: 1.1656x; 1.1656x over previous
import jax
import jax.numpy as jnp
from jax.experimental import pallas as pl
from jax.experimental.pallas import tpu as pltpu

NQ = 50000
NW = 50000
V = 100000
E = 800000
B = 128
D_IN = 50
D_HID = 32
D_OUT = 32


def _final_loss_kernel(anchor_ref, pos_ref, click_ref, W0_ref, b0_ref, W1_ref, b1_ref, out_ref):
    anchor = anchor_ref[...]
    pos = pos_ref[...]
    W0a = W0_ref[:D_OUT, :]
    W0b = W0_ref[D_OUT:, :]
    b0 = b0_ref[...]
    W1 = W1_ref[...]
    b1 = b1_ref[...]

    z1 = jnp.maximum(anchor @ W0a + pos @ W0b + b0[None, :], 0.0)
    s1 = jax.nn.sigmoid(z1 @ W1 + b1[None, :])

    Aa = anchor @ W0a
    Ab = anchor @ W0b
    z2 = jnp.maximum(Aa[:, None, :] + Ab[None, :, :] + b0[None, None, :], 0.0)
    s2_ = jax.nn.sigmoid(z2.reshape(B * B, D_OUT) @ W1 + b1[None, :]).reshape(B, B)
    res = s2_ * click_ref[...].astype(jnp.float32)
    s2 = jnp.max(res, axis=1)

    eps = 1e-12
    p1 = jnp.clip(s1[:, 0], eps, 1.0 - eps)
    p2 = jnp.clip(s2, eps, 1.0 - eps)
    loss = -jnp.mean(jnp.log(p1)) - jnp.mean(jnp.log(1.0 - p2))
    out_ref[...] = jnp.reshape(loss, (1, 1))


def _final_loss(anchor, pos, click, D_W0, D_b0, D_W1, D_b1):
    out = pl.pallas_call(
        _final_loss_kernel,
        out_shape=jax.ShapeDtypeStruct((1, 1), jnp.float32),
        in_specs=[
            pl.BlockSpec((B, D_OUT), lambda: (0, 0)),
            pl.BlockSpec((B, D_OUT), lambda: (0, 0)),
            pl.BlockSpec((B, B), lambda: (0, 0)),
            pl.BlockSpec((2 * D_OUT, 32), lambda: (0, 0)),
            pl.BlockSpec((32,), lambda: (0,)),
            pl.BlockSpec((32, 1), lambda: (0, 0)),
            pl.BlockSpec((1,), lambda: (0,)),
        ],
        out_specs=pl.BlockSpec((1, 1), lambda: (0, 0)),
    )(anchor, pos, click, D_W0, D_b0, D_W1, D_b1)
    return out[0, 0]


def _encode_jnp(word_id, edge_side, edge_upd, gid, wordemb, W1_side, b1_side, b1_upd, W2_upd, b2_upd):
    src_s, dst_s = edge_side[0], edge_side[1]
    src_u, dst_u = edge_upd[0], edge_upd[1]
    deg_out_s = jnp.maximum(jnp.zeros((NW,), jnp.float32).at[src_s].add(1.0), 1.0)
    deg_in_s = jnp.maximum(jnp.zeros((NQ,), jnp.float32).at[dst_s].add(1.0), 1.0)
    deg_out_u = jnp.maximum(jnp.zeros((NQ,), jnp.float32).at[src_u].add(1.0), 1.0)
    deg_in_u = jnp.maximum(jnp.zeros((NQ,), jnp.float32).at[dst_u].add(1.0), 1.0)

    word_feat = jnp.take(wordemb, word_id, axis=0)
    h = (word_feat * (deg_out_s ** -0.5)[:, None]) @ W1_side
    agg = jnp.zeros((NQ, D_HID), jnp.float32).at[dst_s].add(h[src_s])
    side_out = agg * (deg_in_s ** -0.5)[:, None] + b1_side
    h1 = jnp.maximum(side_out + b1_upd, 0.0)

    h2p = (h1 * (deg_out_u ** -0.5)[:, None]) @ W2_upd
    agg2 = jnp.zeros((NQ, D_OUT), jnp.float32).at[dst_u].add(h2p[src_u])
    h2 = agg2 * (deg_in_u ** -0.5)[:, None] + b2_upd

    seg_sum = jax.ops.segment_sum(h2, gid, num_segments=B)
    cnt = jnp.maximum(jax.ops.segment_sum(jnp.ones((NQ,), jnp.float32), gid, num_segments=B), 1.0)
    return seg_sum / cnt[:, None]


def kernel(word_id_a, img_emb_a, edge_side_a, edge_upd_a, query_gid_a, word_id_p, img_emb_p, edge_side_p, edge_upd_p, query_gid_p, click_reverse, wordemb, trans_W, trans_b, W1_side, b1_side, W1_upd, b1_upd, W2_upd, b2_upd, D_W0, D_b0, D_W1, D_b1):
    anchor = _encode_jnp(word_id_a, edge_side_a, edge_upd_a, query_gid_a, wordemb, W1_side, b1_side, b1_upd, W2_upd, b2_upd)
    pos = _encode_jnp(word_id_p, edge_side_p, edge_upd_p, query_gid_p, wordemb, W1_side, b1_side, b1_upd, W2_upd, b2_upd)
    return _final_loss(anchor, pos, click_reverse, D_W0, D_b0, D_W1, D_b1)


# speedup vs baseline: 4.9075x; 4.2103x over previous
import functools

import jax
import jax.numpy as jnp
from jax import lax
from jax.experimental import pallas as pl
from jax.experimental.pallas import tpu as pltpu
from jax.experimental.pallas import tpu_sc as plsc

NQ = 50000
NW = 50000
V = 100000
E = 800000
B = 128
D_IN = 50
D_HID = 32
D_OUT = 32

NC = 2
NS = 16
NWK = NC * NS

CH = 128
W_EDGES = (E // (NWK * CH)) * CH
N_CHUNK = W_EDGES // CH
TAIL_BASE = W_EDGES * NWK
N_TAIL = (E - TAIL_BASE) // CH

G_CHUNKS = NQ // CH
G_PER_W = G_CHUNKS // NWK
G_EXTRA = G_CHUNKS - G_PER_W * NWK
G_TAIL = NQ - G_CHUNKS * CH

STRIPE = NQ // NS

_mesh = plsc.VectorSubcoreMesh(core_axis_name="c", subcore_axis_name="s")


def _sc_pre_body(es_a, eu_a, es_p, eu_p, wid_a, wid_p, we,
                 g1a_out, g1p_out, cnt_out,
                 idx_v, rows_v, ones_v, zb1, sem,
                 c_os_a, c_is_a, c_ou_a, c_iu_a, c_os_p, c_is_p, c_ou_p, c_iu_p):
    c = lax.axis_index("c")
    s = lax.axis_index("s")
    w = s * NC + c
    accs = (c_os_a, c_is_a, c_ou_a, c_iu_a, c_os_p, c_is_p, c_ou_p, c_iu_p)

    def _init_ones(i, _):
        ones_v[pl.ds(i * 16, 16)] = jnp.ones((16,), jnp.float32)
        return 0
    lax.fori_loop(0, CH // 16, _init_ones, 0)

    def _init_zb1(i, _):
        zb1[pl.ds(i * 16, 16)] = jnp.zeros((16,), jnp.float32)
        return 0
    lax.fori_loop(0, 3200 // 16, _init_zb1, 0)

    for acc in accs:
        @pl.when(s < NS - 1)
        def _():
            pltpu.sync_copy(zb1, acc.at[pl.ds(s * 3200, 3200)])

        @pl.when(s == NS - 1)
        def _():
            pltpu.sync_copy(zb1.at[pl.ds(0, 2000)], acc.at[pl.ds(48000, 2000)])
    plsc.subcore_barrier()

    def _count_chunk(edges, acc_src, acc_dst, base):
        pltpu.sync_copy(edges.at[0, pl.ds(base, CH)], idx_v)
        pltpu.sync_copy(ones_v, acc_src.at[idx_v], add=True)
        pltpu.sync_copy(edges.at[1, pl.ds(base, CH)], idx_v)
        pltpu.sync_copy(ones_v, acc_dst.at[idx_v], add=True)

    for edges, acc_src, acc_dst in (
        (es_a, c_os_a, c_is_a), (eu_a, c_ou_a, c_iu_a),
        (es_p, c_os_p, c_is_p), (eu_p, c_ou_p, c_iu_p),
    ):
        def _cbody(j, _, edges=edges, acc_src=acc_src, acc_dst=acc_dst):
            _count_chunk(edges, acc_src, acc_dst, w * W_EDGES + j * CH)
            return 0
        lax.fori_loop(0, N_CHUNK, _cbody, 0)

        @pl.when(w < N_TAIL)
        def _(edges=edges, acc_src=acc_src, acc_dst=acc_dst):
            _count_chunk(edges, acc_src, acc_dst, TAIL_BASE + w * CH)

    def _gather_chunk(wid_ref, out_ref, base, n):
        pltpu.sync_copy(wid_ref.at[pl.ds(base, n)], idx_v.at[pl.ds(0, n)])
        pltpu.async_copy(we.at[idx_v.at[pl.ds(0, n)]], rows_v.at[pl.ds(0, n)], sem).wait()
        pltpu.sync_copy(rows_v.at[pl.ds(0, n)], out_ref.at[pl.ds(base, n)])

    for wid_ref, out_ref in ((wid_a, g1a_out), (wid_p, g1p_out)):
        def _gbody(j, _, wid_ref=wid_ref, out_ref=out_ref):
            _gather_chunk(wid_ref, out_ref, (w + NWK * j) * CH, CH)
            return 0
        lax.fori_loop(0, G_PER_W, _gbody, 0)

        @pl.when(w < G_EXTRA)
        def _(wid_ref=wid_ref, out_ref=out_ref):
            _gather_chunk(wid_ref, out_ref, (G_PER_W * NWK + w) * CH, CH)

        @pl.when(w == G_EXTRA)
        def _(wid_ref=wid_ref, out_ref=out_ref):
            _gather_chunk(wid_ref, out_ref, G_CHUNKS * CH, G_TAIL)

    plsc.subcore_barrier()
    for k, acc in enumerate(accs):
        @pl.when(s < NS - 1)
        def _(k=k, acc=acc):
            pltpu.sync_copy(acc.at[pl.ds(s * 3200, 3200)], zb1)
            pltpu.sync_copy(zb1, cnt_out.at[c, k, pl.ds(s * 3200, 3200)])

        @pl.when(s == NS - 1)
        def _(k=k, acc=acc):
            pltpu.sync_copy(acc.at[pl.ds(48000, 2000)], zb1.at[pl.ds(0, 2000)])
            pltpu.sync_copy(zb1.at[pl.ds(0, 2000)],
                            cnt_out.at[c, k, pl.ds(48000, 2000)])


def _sc_pre(es_a, eu_a, es_p, eu_p, wid_a, wid_p, we):
    return pl.kernel(
        _sc_pre_body,
        out_type=(
            jax.ShapeDtypeStruct((NQ, D_HID), jnp.float32),
            jax.ShapeDtypeStruct((NQ, D_HID), jnp.float32),
            jax.ShapeDtypeStruct((NC, 8, NQ), jnp.float32),
        ),
        mesh=_mesh,
        compiler_params=pltpu.CompilerParams(use_tc_tiling_on_sc=False),
        scratch_types=[
            pltpu.VMEM((CH,), jnp.int32),
            pltpu.VMEM((CH, D_HID), jnp.float32),
            pltpu.VMEM((CH,), jnp.float32),
            pltpu.VMEM((3200,), jnp.float32),
            pltpu.SemaphoreType.DMA,
        ] + [pltpu.VMEM_SHARED((NQ,), jnp.float32)] * 8,
        name="sc_pre",
    )(es_a, eu_a, es_p, eu_p, wid_a, wid_p, we)


DH = D_HID // 2
ZROWS = 125
NRCH = N_CHUNK + 1


def _sc_edge_body(y_a0, y_a1, y_p0, y_p1, ed_a, ed_p,
                  oa0, oa1, op0, op1,
                  idxs, rows_v, zb, bb, sem, acc):
    c = lax.axis_index("c")
    s = lax.axis_index("s")
    w = s * NC + c

    def _init_zb(i, _):
        zb[i, pl.ds(0, 16)] = jnp.zeros((16,), jnp.float32)
        return 0
    lax.fori_loop(0, ZROWS, _init_zb, 0)

    def _zero_acc():
        def _zero_chunk(k, _):
            pltpu.sync_copy(zb, acc.at[pl.ds(s * STRIPE + k * ZROWS, ZROWS)])
            return 0
        lax.fori_loop(0, STRIPE // ZROWS, _zero_chunk, 0)

    def _scatter_pass(y):
        def _chunk(j, _):
            pltpu.async_copy(y.at[idxs.at[0, j]], rows_v, sem).wait()
            pltpu.sync_copy(rows_v, acc.at[idxs.at[1, j]], add=True)
            return 0
        lax.fori_loop(0, N_CHUNK, _chunk, 0)

        @pl.when(w < N_TAIL)
        def _():
            pltpu.async_copy(y.at[idxs.at[0, N_CHUNK]], rows_v, sem).wait()
            pltpu.sync_copy(rows_v, acc.at[idxs.at[1, N_CHUNK]], add=True)

    def _writeback(out):
        def _wb_chunk(k, _):
            base = s * STRIPE + k * ZROWS
            pltpu.sync_copy(acc.at[pl.ds(base, ZROWS)], bb)
            pltpu.sync_copy(bb, out.at[c, pl.ds(base, ZROWS)])
            return 0
        lax.fori_loop(0, STRIPE // ZROWS, _wb_chunk, 0)

    for y0, y1, edges, out0, out1 in ((y_a0, y_a1, ed_a, oa0, oa1),
                                      (y_p0, y_p1, ed_p, op0, op1)):
        _zero_acc()

        def _load_idx(j, _, edges=edges):
            base = w * W_EDGES + j * CH
            pltpu.sync_copy(edges.at[0, pl.ds(base, CH)], idxs.at[0, j])
            pltpu.sync_copy(edges.at[1, pl.ds(base, CH)], idxs.at[1, j])
            return 0
        lax.fori_loop(0, N_CHUNK, _load_idx, 0)

        @pl.when(w < N_TAIL)
        def _(edges=edges):
            base = TAIL_BASE + w * CH
            pltpu.sync_copy(edges.at[0, pl.ds(base, CH)], idxs.at[0, N_CHUNK])
            pltpu.sync_copy(edges.at[1, pl.ds(base, CH)], idxs.at[1, N_CHUNK])

        plsc.subcore_barrier()
        _scatter_pass(y0)
        plsc.subcore_barrier()
        _writeback(out0)
        _zero_acc()
        plsc.subcore_barrier()
        _scatter_pass(y1)
        plsc.subcore_barrier()
        _writeback(out1)


def _sc_edge(y_a0, y_a1, y_p0, y_p1, ed_a, ed_p, name):
    return pl.kernel(
        _sc_edge_body,
        out_type=tuple(
            jax.ShapeDtypeStruct((NC, NQ, DH), jnp.float32) for _ in range(4)),
        mesh=_mesh,
        compiler_params=pltpu.CompilerParams(use_tc_tiling_on_sc=False),
        scratch_types=[
            pltpu.VMEM((2, NRCH, CH), jnp.int32),
            pltpu.VMEM((CH, DH), jnp.float32),
            pltpu.VMEM((ZROWS, DH), jnp.float32),
            pltpu.VMEM((ZROWS, DH), jnp.float32),
            pltpu.SemaphoreType.DMA,
            pltpu.VMEM_SHARED((NQ, DH), jnp.float32),
        ],
        name=name,
    )(y_a0, y_a1, y_p0, y_p1, ed_a, ed_p)


RB = 1000
NRB = NQ // RB


def _we_matmul_body(we_ref, w1_ref, out_ref):
    out_ref[...] = jnp.dot(we_ref[...], w1_ref[...],
                           preferred_element_type=jnp.float32)


def _we_matmul(wordemb, W1_side):
    blk = 2000
    return pl.pallas_call(
        _we_matmul_body,
        grid=(V // blk,),
        in_specs=[
            pl.BlockSpec((blk, D_IN), lambda i: (i, 0)),
            pl.BlockSpec((D_IN, D_HID), lambda i: (0, 0)),
        ],
        out_specs=pl.BlockSpec((blk, D_HID), lambda i: (i, 0)),
        out_shape=jax.ShapeDtypeStruct((V, D_HID), jnp.float32),
        name="tc_we_matmul",
    )(wordemb, W1_side)


def _scale_body(g1a_ref, g1p_ref, cnt_ref,
                ya0_ref, ya1_ref, yp0_ref, yp1_ref, sc_ref):
    cnt = cnt_ref[...]
    sc = lax.rsqrt(jnp.maximum(cnt[:, 0, :] + cnt[:, 1, :], 1.0))
    sc_ref[...] = sc
    ya = g1a_ref[...] * sc[:, 0:1]
    yp = g1p_ref[...] * sc[:, 4:5]
    ya0_ref[...] = ya[:, :DH]
    ya1_ref[...] = ya[:, DH:]
    yp0_ref[...] = yp[:, :DH]
    yp1_ref[...] = yp[:, DH:]


def _scale(g1a, g1p, cntT):
    return pl.pallas_call(
        _scale_body,
        grid=(NRB,),
        in_specs=[
            pl.BlockSpec((RB, D_HID), lambda i: (i, 0)),
            pl.BlockSpec((RB, D_HID), lambda i: (i, 0)),
            pl.BlockSpec((RB, NC, 8), lambda i: (i, 0, 0)),
        ],
        out_specs=[pl.BlockSpec((RB, DH), lambda i: (i, 0))] * 4 + [
            pl.BlockSpec((RB, 8), lambda i: (i, 0)),
        ],
        out_shape=[jax.ShapeDtypeStruct((NQ, DH), jnp.float32)] * 4 + [
            jax.ShapeDtypeStruct((NQ, 8), jnp.float32),
        ],
        name="tc_scale",
    )(g1a, g1p, cntT)


def _mid_body(aa0_ref, aa1_ref, ap0_ref, ap1_ref, sc_ref,
              b1s_ref, b1u_ref, w2_ref,
              ya0_ref, ya1_ref, yp0_ref, yp1_ref):
    sc = sc_ref[...]
    b = (b1s_ref[...] + b1u_ref[...])[None, :]
    w2 = w2_ref[...]
    for a0, a1, y0_ref, y1_ref, k_is, k_ou in (
            (aa0_ref, aa1_ref, ya0_ref, ya1_ref, 1, 2),
            (ap0_ref, ap1_ref, yp0_ref, yp1_ref, 5, 6)):
        ag = jnp.concatenate([a0[0] + a0[1], a1[0] + a1[1]], axis=1)
        h1 = jnp.maximum(ag * sc[:, k_is:k_is + 1] + b, 0.0)
        y2 = jnp.dot(h1, w2, preferred_element_type=jnp.float32)
        y2 = y2 * sc[:, k_ou:k_ou + 1]
        y0_ref[...] = y2[:, :DH]
        y1_ref[...] = y2[:, DH:]


def _mid(agg_a, agg_p, scales, b1_side, b1_upd, W2_upd):
    return pl.pallas_call(
        _mid_body,
        grid=(NRB,),
        in_specs=[pl.BlockSpec((NC, RB, DH), lambda i: (0, i, 0))] * 4 + [
            pl.BlockSpec((RB, 8), lambda i: (i, 0)),
            pl.BlockSpec((D_HID,), lambda i: (0,)),
            pl.BlockSpec((D_HID,), lambda i: (0,)),
            pl.BlockSpec((D_HID, D_OUT), lambda i: (0, 0)),
        ],
        out_specs=[pl.BlockSpec((RB, DH), lambda i: (i, 0))] * 4,
        out_shape=[jax.ShapeDtypeStruct((NQ, DH), jnp.float32)] * 4,
        name="tc_mid",
    )(*agg_a, *agg_p, scales, b1_side, b1_upd, W2_upd)


def _final_body(aa0_ref, aa1_ref, ap0_ref, ap1_ref, sc_ref,
                gida_ref, gidp_ref, b2_ref,
                click_ref, W0_ref, b0_ref, W1_ref, b1_ref, out_ref,
                acc_a, cnt_a, acc_p, cnt_p):
    i = pl.program_id(0)

    @pl.when(i == 0)
    def _():
        acc_a[...] = jnp.zeros((B, D_OUT), jnp.float32)
        cnt_a[...] = jnp.zeros((B, 1), jnp.float32)
        acc_p[...] = jnp.zeros((B, D_OUT), jnp.float32)
        cnt_p[...] = jnp.zeros((B, 1), jnp.float32)

    sc = sc_ref[...]
    b2 = b2_ref[...][None, :]
    iota = lax.broadcasted_iota(jnp.int32, (B, RB), 0)
    ones_col = jnp.ones((RB, 1), jnp.float32)
    for a0, a1, gid_ref, acc, cnt, k_iu in (
            (aa0_ref, aa1_ref, gida_ref, acc_a, cnt_a, 3),
            (ap0_ref, ap1_ref, gidp_ref, acc_p, cnt_p, 7)):
        ag = jnp.concatenate([a0[0] + a0[1], a1[0] + a1[1]], axis=1)
        h2 = ag * sc[:, k_iu:k_iu + 1] + b2
        gid = gid_ref[0, 0]
        mask = (gid[None, :] == iota).astype(jnp.float32)
        acc[...] += jnp.dot(mask, h2, preferred_element_type=jnp.float32)
        cnt[...] += jnp.dot(mask, ones_col, preferred_element_type=jnp.float32)

    @pl.when(i == NRB - 1)
    def _():
        anchor = acc_a[...] / jnp.maximum(cnt_a[...], 1.0)
        pos = acc_p[...] / jnp.maximum(cnt_p[...], 1.0)
        W0a = W0_ref[:D_OUT, :]
        W0b = W0_ref[D_OUT:, :]
        b0 = b0_ref[...][None, :]
        W1 = W1_ref[...]
        b1 = b1_ref[...][None, :]
        z1 = jnp.maximum(anchor @ W0a + pos @ W0b + b0, 0.0)
        s1 = jax.nn.sigmoid(z1 @ W1 + b1)
        Aa = anchor @ W0a
        Ab = anchor @ W0b
        z2 = jnp.maximum(Aa[:, None, :] + Ab[None, :, :] + b0[None, :, :], 0.0)
        s2_ = jax.nn.sigmoid(
            z2.reshape(B * B, D_OUT) @ W1 + b1).reshape(B, B)
        res = s2_ * click_ref[...].astype(jnp.float32)
        s2 = jnp.max(res, axis=1)
        eps = 1e-12
        p1 = jnp.clip(s1[:, 0], eps, 1.0 - eps)
        p2 = jnp.clip(s2, eps, 1.0 - eps)
        loss = -jnp.mean(jnp.log(p1)) - jnp.mean(jnp.log(1.0 - p2))
        out_ref[...] = jnp.reshape(loss, (1, 1))


def _final(agg2_a, agg2_p, scales, gid_a, gid_p, b2_upd, click,
           D_W0, D_b0, D_W1, D_b1):
    gid_a3 = gid_a.reshape(NRB, 1, RB)
    gid_p3 = gid_p.reshape(NRB, 1, RB)
    return pl.pallas_call(
        _final_body,
        grid=(NRB,),
        in_specs=[pl.BlockSpec((NC, RB, DH), lambda i: (0, i, 0))] * 4 + [
            pl.BlockSpec((RB, 8), lambda i: (i, 0)),
            pl.BlockSpec((1, 1, RB), lambda i: (i, 0, 0)),
            pl.BlockSpec((1, 1, RB), lambda i: (i, 0, 0)),
            pl.BlockSpec((D_OUT,), lambda i: (0,)),
            pl.BlockSpec((B, B), lambda i: (0, 0)),
            pl.BlockSpec((2 * D_OUT, 32), lambda i: (0, 0)),
            pl.BlockSpec((32,), lambda i: (0,)),
            pl.BlockSpec((32, 1), lambda i: (0, 0)),
            pl.BlockSpec((1,), lambda i: (0,)),
        ],
        out_specs=pl.BlockSpec((1, 1), lambda i: (0, 0)),
        out_shape=jax.ShapeDtypeStruct((1, 1), jnp.float32),
        scratch_shapes=[
            pltpu.VMEM((B, D_OUT), jnp.float32),
            pltpu.VMEM((B, 1), jnp.float32),
            pltpu.VMEM((B, D_OUT), jnp.float32),
            pltpu.VMEM((B, 1), jnp.float32),
        ],
        name="tc_final",
    )(*agg2_a, *agg2_p, scales, gid_a3, gid_p3, b2_upd, click,
      D_W0, D_b0, D_W1, D_b1)


def kernel(word_id_a, img_emb_a, edge_side_a, edge_upd_a, query_gid_a, word_id_p, img_emb_p, edge_side_p, edge_upd_p, query_gid_p, click_reverse, wordemb, trans_W, trans_b, W1_side, b1_side, W1_upd, b1_upd, W2_upd, b2_upd, D_W0, D_b0, D_W1, D_b1):
    we = _we_matmul(wordemb, W1_side)
    g1a, g1p, cnt = _sc_pre(edge_side_a, edge_upd_a, edge_side_p, edge_upd_p,
                            word_id_a, word_id_p, we)
    cntT = jnp.transpose(cnt, (2, 0, 1))
    ya0, ya1, yp0, yp1, scales = _scale(g1a, g1p, cntT)
    oa0, oa1, op0, op1 = _sc_edge(ya0, ya1, yp0, yp1,
                                  edge_side_a, edge_side_p, "sc_edge1")
    y2a0, y2a1, y2p0, y2p1 = _mid(((oa0, oa1)), ((op0, op1)), scales,
                                  b1_side, b1_upd, W2_upd)
    o2a0, o2a1, o2p0, o2p1 = _sc_edge(y2a0, y2a1, y2p0, y2p1,
                                      edge_upd_a, edge_upd_p, "sc_edge2")
    agg2_a = (o2a0, o2a1)
    agg2_p = (o2p0, o2p1)
    loss = _final(agg2_a, agg2_p, scales, query_gid_a, query_gid_p, b2_upd,
                  click_reverse, D_W0, D_b0, D_W1, D_b1)
    return jnp.reshape(loss, ())


# speedup vs baseline: 17.5719x; 3.5806x over previous
import jax
import jax.numpy as jnp
from jax import lax
from jax.experimental import pallas as pl
from jax.experimental.pallas import tpu as pltpu
from jax.experimental.pallas import tpu_sc as plsc

NQ = 50000
NW = 50000
V = 100000
E = 800000
B = 128
D_IN = 50
D_HID = 32
D_OUT = 32

NC = 2
NS = 16
NWK = NC * NS

CH = 128
ROWS_E = E // CH
RPW = ROWS_E // NWK
TAILR = RPW * NWK
NJ = 200
NBUF = 8
NQ2 = 51200
DUMP_DST = 50048
DUMP_SRC = 50560

G_ROWS = 416
GPW = G_ROWS // NWK
NQG = G_ROWS * CH

STRIPE = NQ // NS

_mesh = plsc.VectorSubcoreMesh(core_axis_name="c", subcore_axis_name="s")
_sc_params = pltpu.CompilerParams(use_tc_tiling_on_sc=False)


def _dummy_row(idxs, plane, r, base):
    for t in range(CH // 16):
        idxs[plane, r, pl.ds(16 * t, 16)] = (
            base + 16 * t + lax.iota(jnp.int32, 16))


def _sc_pre_body(es_a, eu_a, es_p, eu_p, wid_a, wid_p, we,
                 g1a_out, g1p_out, cnt_out,
                 idxs, gidx, rows_v, ones_v, zb1, sem, sem_s,
                 c_os_a, c_is_a, c_ou_a, c_iu_a, c_os_p, c_is_p, c_ou_p, c_iu_p):
    c = lax.axis_index("c")
    s = lax.axis_index("s")
    w = s * NC + c
    accs = (c_os_a, c_is_a, c_ou_a, c_iu_a, c_os_p, c_is_p, c_ou_p, c_iu_p)

    def _init_ones(i, _):
        ones_v[pl.ds(i * 16, 16)] = jnp.ones((16,), jnp.float32)
        return 0
    lax.fori_loop(0, CH // 16, _init_ones, 0)

    def _init_zb1(i, _):
        zb1[pl.ds(i * 16, 16)] = jnp.zeros((16,), jnp.float32)
        return 0
    lax.fori_loop(0, 3200 // 16, _init_zb1, 0)

    for acc in accs:
        pltpu.sync_copy(zb1, acc.at[pl.ds(s * 3200, 3200)])
    plsc.subcore_barrier()

    for edges, acc_src, acc_dst in (
        (es_a, c_os_a, c_is_a), (eu_a, c_ou_a, c_iu_a),
        (es_p, c_os_p, c_is_p), (eu_p, c_ou_p, c_iu_p),
    ):
        for k in (0, 1):
            pltpu.sync_copy(edges.at[k, pl.ds(w * RPW, RPW)],
                            idxs.at[k, pl.ds(0, RPW)])

        @pl.when(w < ROWS_E - TAILR)
        def _(edges=edges):
            for k in (0, 1):
                pltpu.sync_copy(edges.at[k, TAILR + w], idxs.at[k, RPW])

        @pl.when(w >= ROWS_E - TAILR)
        def _():
            _dummy_row(idxs, 0, RPW, DUMP_SRC)
            _dummy_row(idxs, 1, RPW, DUMP_DST)

        def _grp(g, _, acc_src=acc_src, acc_dst=acc_dst):
            for b in range(4):
                j = g * 4 + b
                pltpu.async_copy(ones_v, acc_src.at[idxs.at[0, j]], sem_s,
                                 add=True)
                pltpu.async_copy(ones_v, acc_dst.at[idxs.at[1, j]], sem_s,
                                 add=True)
            for b in range(4):
                j = g * 4 + b
                pltpu.make_async_copy(ones_v, acc_src.at[idxs.at[0, j]],
                                      sem_s).wait()
                pltpu.make_async_copy(ones_v, acc_dst.at[idxs.at[1, j]],
                                      sem_s).wait()
            return 0
        lax.fori_loop(0, (RPW + 1) // 4, _grp, 0)

    for wid_ref, out_ref in ((wid_a, g1a_out), (wid_p, g1p_out)):
        pltpu.sync_copy(wid_ref.at[pl.ds(w * GPW, GPW)], gidx)

        def _gbody(j, _, out_ref=out_ref):
            base = (w * GPW + j) * CH
            pltpu.async_copy(we.at[gidx.at[j]], rows_v, sem).wait()
            pltpu.sync_copy(rows_v, out_ref.at[pl.ds(base, CH)])
            return 0
        lax.fori_loop(0, GPW, _gbody, 0)

    plsc.subcore_barrier()
    for k, acc in enumerate(accs):
        @pl.when(s < NS - 1)
        def _(k=k, acc=acc):
            pltpu.sync_copy(acc.at[pl.ds(s * 3200, 3200)], zb1)
            pltpu.sync_copy(zb1, cnt_out.at[c, k, pl.ds(s * 3200, 3200)])

        @pl.when(s == NS - 1)
        def _(k=k, acc=acc):
            pltpu.sync_copy(acc.at[pl.ds(48000, 2000)], zb1.at[pl.ds(0, 2000)])
            pltpu.sync_copy(zb1.at[pl.ds(0, 2000)],
                            cnt_out.at[c, k, pl.ds(48000, 2000)])


def _sc_pre(es_a, eu_a, es_p, eu_p, wid_a, wid_p, we):
    return pl.kernel(
        _sc_pre_body,
        out_type=(
            jax.ShapeDtypeStruct((NQG, D_HID), jnp.float32),
            jax.ShapeDtypeStruct((NQG, D_HID), jnp.float32),
            jax.ShapeDtypeStruct((NC, 8, NQ), jnp.float32),
        ),
        mesh=_mesh,
        compiler_params=_sc_params,
        scratch_types=[
            pltpu.VMEM((2, RPW + 1, CH), jnp.int32),
            pltpu.VMEM((GPW, CH), jnp.int32),
            pltpu.VMEM((CH, D_HID), jnp.float32),
            pltpu.VMEM((CH,), jnp.float32),
            pltpu.VMEM((3200,), jnp.float32),
            pltpu.SemaphoreType.DMA,
            pltpu.SemaphoreType.DMA,
        ] + [pltpu.VMEM_SHARED((NQ2,), jnp.float32)] * 8,
        name="sc_pre",
    )(es_a, eu_a, es_p, eu_p, wid_a, wid_p, we)


def _sc_edge_body(y_a, y_p, ed_a, ed_p,
                  out_a, out_p,
                  idxs, rows, zb, bb, sem_g, sem_s, acc):
    c = lax.axis_index("c")
    s = lax.axis_index("s")
    w = s * NC + c

    def _init_zb(i, _):
        zb[i, :] = jnp.zeros((D_HID,), jnp.bfloat16)
        return 0
    lax.fori_loop(0, CH, _init_zb, 0)

    for y, edges, out in ((y_a, ed_a, out_a), (y_p, ed_p, out_p)):
        for k in (0, 1):
            pltpu.sync_copy(edges.at[k, pl.ds(w * RPW, RPW)],
                            idxs.at[k, pl.ds(0, RPW)])

        @pl.when(w < ROWS_E - TAILR)
        def _(edges=edges):
            for k in (0, 1):
                pltpu.sync_copy(edges.at[k, TAILR + w], idxs.at[k, RPW])

        @pl.when(w >= ROWS_E - TAILR)
        def _():
            _dummy_row(idxs, 0, RPW, 0)
            _dummy_row(idxs, 1, RPW, DUMP_DST)
        for r in range(RPW + 1, NJ):
            _dummy_row(idxs, 0, r, 0)
            _dummy_row(idxs, 1, r, DUMP_DST)

        def _zero_chunk(k, _):
            pltpu.sync_copy(zb, acc.at[pl.ds(s * 3200 + k * CH, CH)])
            return 0
        lax.fori_loop(0, 3200 // CH, _zero_chunk, 0)
        plsc.subcore_barrier()

        for j in range(4):
            pltpu.async_copy(y.at[idxs.at[0, j]], rows[j], sem_g[j])

        def _grp(g, _, y=y):
            for b in range(NBUF):
                j = g * NBUF + b
                pltpu.make_async_copy(y.at[idxs.at[0, j]], rows[b],
                                      sem_g[b]).wait()
                pltpu.async_copy(rows[b], acc.at[idxs.at[1, j]], sem_s[b],
                                 add=True)
                jn = j + 4
                bn = (b + 4) % NBUF

                @pl.when(jn < NJ)
                def _(jn=jn, bn=bn, y=y):
                    @pl.when(jn >= NBUF)
                    def _():
                        pltpu.make_async_copy(
                            rows[bn], acc.at[idxs.at[1, jn - NBUF]],
                            sem_s[bn]).wait()
                    pltpu.async_copy(y.at[idxs.at[0, jn]], rows[bn],
                                     sem_g[bn])
            return 0
        lax.fori_loop(0, NJ // NBUF, _grp, 0)

        for j in range(NJ - NBUF, NJ):
            b = j % NBUF
            pltpu.make_async_copy(rows[b], acc.at[idxs.at[1, j]],
                                  sem_s[b]).wait()

        plsc.subcore_barrier()

        def _wb_chunk(k, _, out=out):
            base = s * STRIPE + k * 125
            pltpu.sync_copy(acc.at[pl.ds(base, 125)], bb)
            pltpu.sync_copy(bb, out.at[c, pl.ds(base, 125)])
            return 0
        lax.fori_loop(0, STRIPE // 125, _wb_chunk, 0)


def _sc_edge(y_a, y_p, ed_a, ed_p, name):
    return pl.kernel(
        _sc_edge_body,
        out_type=tuple(
            jax.ShapeDtypeStruct((NC, NQ, D_HID), jnp.bfloat16)
            for _ in range(2)),
        mesh=_mesh,
        compiler_params=_sc_params,
        scratch_types=[
            pltpu.VMEM((2, NJ, CH), jnp.int32),
            [pltpu.VMEM((CH, D_HID), jnp.bfloat16) for _ in range(NBUF)],
            pltpu.VMEM((CH, D_HID), jnp.bfloat16),
            pltpu.VMEM((125, D_HID), jnp.bfloat16),
            [pltpu.SemaphoreType.DMA for _ in range(NBUF)],
            [pltpu.SemaphoreType.DMA for _ in range(NBUF)],
            pltpu.VMEM_SHARED((NQ2, D_HID), jnp.bfloat16),
        ],
        name=name,
    )(y_a, y_p, ed_a, ed_p)


RB = 2000
NRB = NQ // RB


def _we_matmul_body(we_ref, w1_ref, out_ref):
    out_ref[...] = jnp.dot(we_ref[...], w1_ref[...],
                           preferred_element_type=jnp.float32)


def _we_matmul(wordemb, W1_side):
    blk = 2000
    return pl.pallas_call(
        _we_matmul_body,
        grid=(V // blk,),
        in_specs=[
            pl.BlockSpec((blk, D_IN), lambda i: (i, 0)),
            pl.BlockSpec((D_IN, D_HID), lambda i: (0, 0)),
        ],
        out_specs=pl.BlockSpec((blk, D_HID), lambda i: (i, 0)),
        out_shape=jax.ShapeDtypeStruct((V, D_HID), jnp.float32),
        name="tc_we_matmul",
    )(wordemb, W1_side)


def _scale_body(g1a_ref, g1p_ref, cnt_ref, ya_ref, yp_ref, sc_ref):
    cnt = cnt_ref[...]
    sc = lax.rsqrt(jnp.maximum(cnt[:, 0, :] + cnt[:, 1, :], 1.0))
    sc_ref[...] = sc
    ya_ref[...] = (g1a_ref[...] * sc[:, 0:1]).astype(jnp.bfloat16)
    yp_ref[...] = (g1p_ref[...] * sc[:, 4:5]).astype(jnp.bfloat16)


def _scale(g1a, g1p, cntT):
    return pl.pallas_call(
        _scale_body,
        grid=(NRB,),
        in_specs=[
            pl.BlockSpec((RB, D_HID), lambda i: (i, 0)),
            pl.BlockSpec((RB, D_HID), lambda i: (i, 0)),
            pl.BlockSpec((RB, NC, 8), lambda i: (i, 0, 0)),
        ],
        out_specs=[pl.BlockSpec((RB, D_HID), lambda i: (i, 0))] * 2 + [
            pl.BlockSpec((RB, 8), lambda i: (i, 0)),
        ],
        out_shape=[jax.ShapeDtypeStruct((NQ, D_HID), jnp.bfloat16)] * 2 + [
            jax.ShapeDtypeStruct((NQ, 8), jnp.float32),
        ],
        name="tc_scale",
    )(g1a, g1p, cntT)


def _mid_body(aga_ref, agp_ref, sc_ref, b1s_ref, b1u_ref, w2_ref,
              ya_ref, yp_ref):
    sc = sc_ref[...]
    b = (b1s_ref[...] + b1u_ref[...])[None, :]
    w2 = w2_ref[...]
    for ag_ref, y_ref, k_is, k_ou in ((aga_ref, ya_ref, 1, 2),
                                      (agp_ref, yp_ref, 5, 6)):
        ag = (ag_ref[0].astype(jnp.float32) + ag_ref[1].astype(jnp.float32))
        h1 = jnp.maximum(ag * sc[:, k_is:k_is + 1] + b, 0.0)
        y2 = jnp.dot(h1, w2, preferred_element_type=jnp.float32)
        y_ref[...] = (y2 * sc[:, k_ou:k_ou + 1]).astype(jnp.bfloat16)


def _mid(agg_a, agg_p, scales, b1_side, b1_upd, W2_upd):
    return pl.pallas_call(
        _mid_body,
        grid=(NRB,),
        in_specs=[pl.BlockSpec((NC, RB, D_HID), lambda i: (0, i, 0))] * 2 + [
            pl.BlockSpec((RB, 8), lambda i: (i, 0)),
            pl.BlockSpec((D_HID,), lambda i: (0,)),
            pl.BlockSpec((D_HID,), lambda i: (0,)),
            pl.BlockSpec((D_HID, D_OUT), lambda i: (0, 0)),
        ],
        out_specs=[pl.BlockSpec((RB, D_HID), lambda i: (i, 0))] * 2,
        out_shape=[jax.ShapeDtypeStruct((NQ, D_HID), jnp.bfloat16)] * 2,
        name="tc_mid",
    )(agg_a, agg_p, scales, b1_side, b1_upd, W2_upd)


def _final_body(aga_ref, agp_ref, sc_ref, gida_ref, gidp_ref, b2_ref,
                click_ref, W0_ref, b0_ref, W1_ref, b1_ref, out_ref,
                acc_a, cnt_a, acc_p, cnt_p):
    i = pl.program_id(0)

    @pl.when(i == 0)
    def _():
        acc_a[...] = jnp.zeros((B, D_OUT), jnp.float32)
        cnt_a[...] = jnp.zeros((B, 1), jnp.float32)
        acc_p[...] = jnp.zeros((B, D_OUT), jnp.float32)
        cnt_p[...] = jnp.zeros((B, 1), jnp.float32)

    sc = sc_ref[...]
    b2 = b2_ref[...][None, :]
    iota = lax.broadcasted_iota(jnp.int32, (B, RB), 0)
    ones_col = jnp.ones((RB, 1), jnp.float32)
    for ag_ref, gid_ref, acc, cnt, k_iu in (
            (aga_ref, gida_ref, acc_a, cnt_a, 3),
            (agp_ref, gidp_ref, acc_p, cnt_p, 7)):
        ag = (ag_ref[0].astype(jnp.float32) + ag_ref[1].astype(jnp.float32))
        h2 = ag * sc[:, k_iu:k_iu + 1] + b2
        gid = gid_ref[0, 0]
        mask = (gid[None, :] == iota).astype(jnp.float32)
        acc[...] += jnp.dot(mask, h2, preferred_element_type=jnp.float32)
        cnt[...] += jnp.dot(mask, ones_col, preferred_element_type=jnp.float32)

    @pl.when(i == NRB - 1)
    def _():
        anchor = acc_a[...] / jnp.maximum(cnt_a[...], 1.0)
        pos = acc_p[...] / jnp.maximum(cnt_p[...], 1.0)
        W0a = W0_ref[:D_OUT, :]
        W0b = W0_ref[D_OUT:, :]
        b0 = b0_ref[...][None, :]
        W1 = W1_ref[...]
        b1 = b1_ref[...][None, :]
        z1 = jnp.maximum(anchor @ W0a + pos @ W0b + b0, 0.0)
        s1 = jax.nn.sigmoid(z1 @ W1 + b1)
        Aa = anchor @ W0a
        Ab = anchor @ W0b
        z2 = jnp.maximum(Aa[:, None, :] + Ab[None, :, :] + b0[None, :, :], 0.0)
        s2_ = jax.nn.sigmoid(
            z2.reshape(B * B, D_OUT) @ W1 + b1).reshape(B, B)
        res = s2_ * click_ref[...].astype(jnp.float32)
        s2 = jnp.max(res, axis=1)
        eps = 1e-12
        p1 = jnp.clip(s1[:, 0], eps, 1.0 - eps)
        p2 = jnp.clip(s2, eps, 1.0 - eps)
        loss = -jnp.mean(jnp.log(p1)) - jnp.mean(jnp.log(1.0 - p2))
        out_ref[...] = jnp.reshape(loss, (1, 1))


def _final(agg2_a, agg2_p, scales, gid_a, gid_p, b2_upd, click,
           D_W0, D_b0, D_W1, D_b1):
    gid_a3 = gid_a.reshape(NRB, 1, RB)
    gid_p3 = gid_p.reshape(NRB, 1, RB)
    return pl.pallas_call(
        _final_body,
        grid=(NRB,),
        in_specs=[pl.BlockSpec((NC, RB, D_HID), lambda i: (0, i, 0))] * 2 + [
            pl.BlockSpec((RB, 8), lambda i: (i, 0)),
            pl.BlockSpec((1, 1, RB), lambda i: (i, 0, 0)),
            pl.BlockSpec((1, 1, RB), lambda i: (i, 0, 0)),
            pl.BlockSpec((D_OUT,), lambda i: (0,)),
            pl.BlockSpec((B, B), lambda i: (0, 0)),
            pl.BlockSpec((2 * D_OUT, 32), lambda i: (0, 0)),
            pl.BlockSpec((32,), lambda i: (0,)),
            pl.BlockSpec((32, 1), lambda i: (0, 0)),
            pl.BlockSpec((1,), lambda i: (0,)),
        ],
        out_specs=pl.BlockSpec((1, 1), lambda i: (0, 0)),
        out_shape=jax.ShapeDtypeStruct((1, 1), jnp.float32),
        scratch_shapes=[
            pltpu.VMEM((B, D_OUT), jnp.float32),
            pltpu.VMEM((B, 1), jnp.float32),
            pltpu.VMEM((B, D_OUT), jnp.float32),
            pltpu.VMEM((B, 1), jnp.float32),
        ],
        name="tc_final",
    )(agg2_a, agg2_p, scales, gid_a3, gid_p3, b2_upd, click,
      D_W0, D_b0, D_W1, D_b1)


def kernel(word_id_a, img_emb_a, edge_side_a, edge_upd_a, query_gid_a, word_id_p, img_emb_p, edge_side_p, edge_upd_p, query_gid_p, click_reverse, wordemb, trans_W, trans_b, W1_side, b1_side, W1_upd, b1_upd, W2_upd, b2_upd, D_W0, D_b0, D_W1, D_b1):
    pad_ids = jnp.arange(NQG - NQ, dtype=jnp.int32) % V
    wid_a = jnp.concatenate([word_id_a, pad_ids]).reshape(G_ROWS, CH)
    wid_p = jnp.concatenate([word_id_p, pad_ids]).reshape(G_ROWS, CH)
    es_a = edge_side_a.reshape(2, ROWS_E, CH)
    eu_a = edge_upd_a.reshape(2, ROWS_E, CH)
    es_p = edge_side_p.reshape(2, ROWS_E, CH)
    eu_p = edge_upd_p.reshape(2, ROWS_E, CH)

    we = _we_matmul(wordemb, W1_side)
    g1a, g1p, cnt = _sc_pre(es_a, eu_a, es_p, eu_p, wid_a, wid_p, we)
    cntT = jnp.transpose(cnt, (2, 0, 1))
    ya, yp, scales = _scale(g1a, g1p, cntT)
    agg_a, agg_p = _sc_edge(ya, yp, es_a, es_p, "sc_edge1")
    y2a, y2p = _mid(agg_a, agg_p, scales, b1_side, b1_upd, W2_upd)
    agg2_a, agg2_p = _sc_edge(y2a, y2p, eu_a, eu_p, "sc_edge2")
    loss = _final(agg2_a, agg2_p, scales, query_gid_a, query_gid_p, b2_upd,
                  click_reverse, D_W0, D_b0, D_W1, D_b1)
    return jnp.reshape(loss, ())
